# baseline (speedup 1.0000x reference)
import functools

import numpy as np

import jax
import jax.numpy as jnp
from jax import lax
from jax.experimental import pallas as pl
from jax.experimental.pallas import tpu as pltpu
from jax.experimental.pallas import tpu_sc as plsc

_EPS = 1e-12
_L = 16

_PERM = np.concatenate([
    np.arange(0, 32, 2), np.arange(1, 32, 2),
    np.arange(32, 64, 2), np.arange(33, 64, 2),
])
_INV_PERM = np.argsort(_PERM)


def _rsqrt(x):
    y = lax.bitcast_convert_type(
        0x5F3759DF - (lax.bitcast_convert_type(x, jnp.int32) >> 1),
        jnp.float32,
    )
    for _ in range(3):
        y = y * (1.5 - 0.5 * x * y * y)
    return y


def kernel(input_ids, item_table, pos_table, ln_gamma, ln_beta):
    B, S = input_ids.shape
    V, H = item_table.shape
    half = S // 2
    K = H // _L
    ids = input_ids.astype(jnp.int32).reshape(B, 2, half)
    t4 = item_table.reshape(V, H // 32, 2, _L).astype(jnp.bfloat16)
    lo = lax.bitcast_convert_type(t4[:, :, 0, :], jnp.uint16).astype(jnp.uint32)
    hi = lax.bitcast_convert_type(t4[:, :, 1, :], jnp.uint16).astype(jnp.uint32)
    tbl16 = lax.bitcast_convert_type(
        lo | (hi << 16), jnp.bfloat16).reshape(V, H)

    info = plsc.get_sparse_core_info()
    NC, NS = info.num_cores, info.num_subcores
    NW = NC * NS
    seq_per_w = B // NW

    mesh = plsc.VectorSubcoreMesh(core_axis_name="c", subcore_axis_name="s")

    @functools.partial(
        pl.kernel,
        out_type=jax.ShapeDtypeStruct((B, S, H), jnp.float32),
        mesh=mesh,
        compiler_params=pltpu.CompilerParams(
            needs_layout_passes=False, use_tc_tiling_on_sc=False),
        scratch_types=[
            pltpu.VMEM((seq_per_w, 2, half), jnp.int32),
            pltpu.VMEM((4, S, H), jnp.bfloat16),
            pltpu.VMEM((2, S, H), jnp.float32),
            pltpu.VMEM((S, H), jnp.float32),
            pltpu.VMEM((H,), jnp.float32),
            pltpu.VMEM((H,), jnp.float32),
            pltpu.SemaphoreType.DMA,
            pltpu.SemaphoreType.DMA,
            pltpu.SemaphoreType.DMA,
            pltpu.SemaphoreType.DMA,
            pltpu.SemaphoreType.DMA,
            pltpu.SemaphoreType.DMA,
        ],
    )
    def emb_ln(ids_hbm, table_hbm, pos_hbm, gamma_hbm, beta_hbm, out_hbm,
               idx_all, grows, obufs, pos_v, gamma_v, beta_v,
               gsem0, gsem1, gsem2, gsem3, osem0, osem1):
        gsem = (gsem0, gsem1, gsem2, gsem3)
        osem = (osem0, osem1)

        wid = lax.axis_index("c") * NS + lax.axis_index("s")
        q0 = wid * seq_per_w

        pltpu.sync_copy(ids_hbm.at[pl.ds(q0, seq_per_w)], idx_all)
        pltpu.sync_copy(pos_hbm.at[pl.ds(0, S)], pos_v)
        pltpu.sync_copy(gamma_hbm, gamma_v)
        pltpu.sync_copy(beta_hbm, beta_v)
        gv = [gamma_v[pl.ds(k * _L, _L)] for k in range(K)]
        bv = [beta_v[pl.ds(k * _L, _L)] for k in range(K)]

        def issue_gather(c, slot):
            buf = grows.at[slot]
            pltpu.async_copy(
                table_hbm.at[idx_all.at[c, 0]],
                buf.at[pl.ds(0, half)], gsem[slot])
            pltpu.async_copy(
                table_hbm.at[idx_all.at[c, 1]],
                buf.at[pl.ds(half, half)], gsem[slot])

        issue_gather(0, 0)
        issue_gather(1, 1)

        def four_seqs(gi, _):
            g = gi * 4
            for b in range(4):
                c = g + b
                ob = b % 2
                gb = grows.at[b]
                obuf = obufs.at[ob]

                @pl.when(c + 2 < seq_per_w)
                def _():
                    issue_gather(c + 2, (b + 2) % 4)

                pltpu.make_async_copy(
                    table_hbm.at[pl.ds(0, S)], gb, gsem[b]).wait()

                @pl.when(c >= 2)
                def _():
                    pltpu.make_async_copy(
                        obuf, out_hbm.at[q0], osem[ob]).wait()

                @plsc.parallel_loop(0, S, 1, unroll=4)
                def per_row(i):
                    x = []
                    for m in range(K // 2):
                        w = plsc.bitcast(
                            gb[i, pl.ds(m * 2 * _L, 2 * _L)], jnp.int32)
                        x.append(lax.bitcast_convert_type(
                            w << 16, jnp.float32))
                        x.append(lax.bitcast_convert_type(
                            w & jnp.int32(-65536), jnp.float32))
                    x = [xk + pos_v[i, pl.ds(k * _L, _L)]
                         for k, xk in enumerate(x)]
                    tot = jnp.sum((x[0] + x[1]) + (x[2] + x[3]))
                    mean = tot * (1.0 / H)
                    d = [xk - mean for xk in x]
                    sq = ((d[0] * d[0] + d[1] * d[1])
                          + (d[2] * d[2] + d[3] * d[3]))
                    var = jnp.sum(sq) * (1.0 / H)
                    r = _rsqrt(var + _EPS)
                    for k in range(K):
                        obuf[i, pl.ds(k * _L, _L)] = d[k] * r * gv[k] + bv[k]

                pltpu.async_copy(obuf, out_hbm.at[q0 + c], osem[ob])
            return ()

        lax.fori_loop(0, seq_per_w // 4, four_seqs, ())
        pltpu.make_async_copy(obufs.at[0], out_hbm.at[q0], osem0).wait()
        pltpu.make_async_copy(obufs.at[1], out_hbm.at[q0], osem1).wait()

    out = emb_ln(ids, tbl16, pos_table, ln_gamma, ln_beta)
    return out


# speedup vs baseline: 1.0760x; 1.0760x over previous
import functools

import jax
import jax.numpy as jnp
from jax import lax
from jax.experimental import pallas as pl
from jax.experimental.pallas import tpu as pltpu
from jax.experimental.pallas import tpu_sc as plsc

_EPS = 1e-12
_L = 16


def _rsqrt(x):
    y = lax.bitcast_convert_type(
        0x5F3759DF - (lax.bitcast_convert_type(x, jnp.int32) >> 1),
        jnp.float32,
    )
    for _ in range(3):
        y = y * (1.5 - 0.5 * x * y * y)
    return y


def kernel(input_ids, item_table, pos_table, ln_gamma, ln_beta):
    B, S = input_ids.shape
    V, H = item_table.shape
    half = S // 2
    K = H // _L
    ids = input_ids.astype(jnp.int32).reshape(B, 2, half)

    info = plsc.get_sparse_core_info()
    NC, NS = info.num_cores, info.num_subcores
    NW = NC * NS
    seq_per_w = B // NW

    mesh = plsc.VectorSubcoreMesh(core_axis_name="c", subcore_axis_name="s")

    @functools.partial(
        pl.kernel,
        out_type=jax.ShapeDtypeStruct((B, S, H), jnp.float32),
        mesh=mesh,
        compiler_params=pltpu.CompilerParams(
            needs_layout_passes=False, use_tc_tiling_on_sc=False),
        scratch_types=[
            pltpu.VMEM((seq_per_w, 2, half), jnp.int32),
            pltpu.VMEM((4, S, H), jnp.float32),
            pltpu.VMEM((2, S, H), jnp.float32),
            pltpu.VMEM((S, H), jnp.float32),
            pltpu.VMEM((H,), jnp.float32),
            pltpu.VMEM((H,), jnp.float32),
            pltpu.SemaphoreType.DMA,
            pltpu.SemaphoreType.DMA,
            pltpu.SemaphoreType.DMA,
            pltpu.SemaphoreType.DMA,
            pltpu.SemaphoreType.DMA,
            pltpu.SemaphoreType.DMA,
        ],
    )
    def emb_ln(ids_hbm, table_hbm, pos_hbm, gamma_hbm, beta_hbm, out_hbm,
               idx_all, grows, obufs, pos_v, gamma_v, beta_v,
               gsem0, gsem1, gsem2, gsem3, osem0, osem1):
        gsem = (gsem0, gsem1, gsem2, gsem3)
        osem = (osem0, osem1)

        wid = lax.axis_index("c") * NS + lax.axis_index("s")
        q0 = wid * seq_per_w

        pltpu.sync_copy(ids_hbm.at[pl.ds(q0, seq_per_w)], idx_all)
        pltpu.sync_copy(pos_hbm.at[pl.ds(0, S)], pos_v)
        pltpu.sync_copy(gamma_hbm, gamma_v)
        pltpu.sync_copy(beta_hbm, beta_v)
        gv = [gamma_v[pl.ds(k * _L, _L)] for k in range(K)]
        bv = [beta_v[pl.ds(k * _L, _L)] for k in range(K)]

        def issue_gather(c, slot):
            buf = grows.at[slot]
            pltpu.async_copy(
                table_hbm.at[idx_all.at[c, 0]],
                buf.at[pl.ds(0, half)], gsem[slot])
            pltpu.async_copy(
                table_hbm.at[idx_all.at[c, 1]],
                buf.at[pl.ds(half, half)], gsem[slot])

        issue_gather(0, 0)
        issue_gather(1, 1)

        def four_seqs(gi, _):
            g = gi * 4
            for b in range(4):
                c = g + b
                ob = b % 2
                gb = grows.at[b]
                obuf = obufs.at[ob]

                @pl.when(c + 2 < seq_per_w)
                def _():
                    issue_gather(c + 2, (b + 2) % 4)

                pltpu.make_async_copy(out_hbm.at[q0], gb, gsem[b]).wait()

                @pl.when(c >= 2)
                def _():
                    pltpu.make_async_copy(
                        obuf, out_hbm.at[q0], osem[ob]).wait()

                @plsc.parallel_loop(0, S, 1, unroll=8)
                def per_row(i):
                    x = [gb[i, pl.ds(k * _L, _L)] + pos_v[i, pl.ds(k * _L, _L)]
                         for k in range(K)]
                    tot = jnp.sum((x[0] + x[1]) + (x[2] + x[3]))
                    mean = tot * (1.0 / H)
                    d = [xk - mean for xk in x]
                    sq = ((d[0] * d[0] + d[1] * d[1])
                          + (d[2] * d[2] + d[3] * d[3]))
                    var = jnp.sum(sq) * (1.0 / H)
                    r = _rsqrt(var + _EPS)
                    for k in range(K):
                        obuf[i, pl.ds(k * _L, _L)] = d[k] * r * gv[k] + bv[k]

                pltpu.async_copy(obuf, out_hbm.at[q0 + c], osem[ob])
            return ()

        lax.fori_loop(0, seq_per_w // 4, four_seqs, ())
        pltpu.make_async_copy(obufs.at[0], out_hbm.at[q0], osem0).wait()
        pltpu.make_async_copy(obufs.at[1], out_hbm.at[q0], osem1).wait()

    out = emb_ln(ids, item_table, pos_table, ln_gamma, ln_beta)
    return out


# speedup vs baseline: 1.1667x; 1.0843x over previous
import functools

import jax
import jax.numpy as jnp
from jax import lax
from jax.experimental import pallas as pl
from jax.experimental.pallas import tpu as pltpu
from jax.experimental.pallas import tpu_sc as plsc

_EPS = 1e-12
_L = 16


def _rsqrt(x):
    y = lax.bitcast_convert_type(
        0x5F3759DF - (lax.bitcast_convert_type(x, jnp.int32) >> 1),
        jnp.float32,
    )
    for _ in range(3):
        y = y * (1.5 - 0.5 * x * y * y)
    return y


def kernel(input_ids, item_table, pos_table, ln_gamma, ln_beta):
    B, S = input_ids.shape
    V, H = item_table.shape
    half = S // 2
    K = H // _L
    ids = input_ids.astype(jnp.int32).reshape(B, 2, half)

    info = plsc.get_sparse_core_info()
    NC, NS = info.num_cores, info.num_subcores
    NW = NC * NS
    seq_per_w = B // NW

    mesh = plsc.VectorSubcoreMesh(core_axis_name="c", subcore_axis_name="s")

    @functools.partial(
        pl.kernel,
        out_type=jax.ShapeDtypeStruct((B, S, H), jnp.float32),
        mesh=mesh,
        compiler_params=pltpu.CompilerParams(
            needs_layout_passes=False, use_tc_tiling_on_sc=False),
        scratch_types=[
            pltpu.VMEM((seq_per_w, 2, half), jnp.int32),
            pltpu.VMEM((4, S, H), jnp.float32),
            pltpu.VMEM((2, S, H), jnp.float32),
            pltpu.VMEM((S, H), jnp.float32),
            pltpu.VMEM((H,), jnp.float32),
            pltpu.VMEM((H,), jnp.float32),
            pltpu.SemaphoreType.DMA,
            pltpu.SemaphoreType.DMA,
            pltpu.SemaphoreType.DMA,
            pltpu.SemaphoreType.DMA,
            pltpu.SemaphoreType.DMA,
            pltpu.SemaphoreType.DMA,
        ],
    )
    def emb_ln(ids_hbm, table_hbm, pos_hbm, gamma_hbm, beta_hbm, out_hbm,
               idx_all, grows, obufs, pos_v, gamma_v, beta_v,
               gsem0, gsem1, gsem2, gsem3, osem0, osem1):
        gsem = (gsem0, gsem1, gsem2, gsem3)
        osem = (osem0, osem1)

        wid = lax.axis_index("c") * NS + lax.axis_index("s")
        q0 = wid * seq_per_w

        pltpu.sync_copy(ids_hbm.at[pl.ds(q0, seq_per_w)], idx_all)
        pltpu.sync_copy(pos_hbm.at[pl.ds(0, S)], pos_v)
        pltpu.sync_copy(gamma_hbm, gamma_v)
        pltpu.sync_copy(beta_hbm, beta_v)
        gv = [gamma_v[pl.ds(k * _L, _L)] for k in range(K)]
        bv = [beta_v[pl.ds(k * _L, _L)] for k in range(K)]

        def issue_gather(c, slot):
            buf = grows.at[slot]
            pltpu.async_copy(
                table_hbm.at[idx_all.at[c, 0]],
                buf.at[pl.ds(0, half)], gsem[slot])
            pltpu.async_copy(
                table_hbm.at[idx_all.at[c, 1]],
                buf.at[pl.ds(half, half)], gsem[slot])

        issue_gather(0, 0)
        issue_gather(1, 1)

        def four_seqs(gi, _):
            g = gi * 4
            for b in range(4):
                c = g + b
                ob = b % 2
                gb = grows.at[b]
                obuf = obufs.at[ob]

                @pl.when(c + 2 < seq_per_w)
                def _():
                    issue_gather(c + 2, (b + 2) % 4)

                pltpu.make_async_copy(out_hbm.at[q0], gb, gsem[b]).wait()

                @pl.when(c >= 2)
                def _():
                    pltpu.make_async_copy(
                        obuf, out_hbm.at[q0], osem[ob]).wait()

                @plsc.parallel_loop(0, S, 1, unroll=4)
                def per_row(i):
                    x = [gb[i, pl.ds(k * _L, _L)] + pos_v[i, pl.ds(k * _L, _L)]
                         for k in range(K)]
                    tot = jnp.sum((x[0] + x[1]) + (x[2] + x[3]))
                    mean = tot * (1.0 / H)
                    d = [xk - mean for xk in x]
                    sq = ((d[0] * d[0] + d[1] * d[1])
                          + (d[2] * d[2] + d[3] * d[3]))
                    var = jnp.sum(sq) * (1.0 / H)
                    r = _rsqrt(var + _EPS)
                    for k in range(K):
                        obuf[i, pl.ds(k * _L, _L)] = d[k] * r * gv[k] + bv[k]

                pltpu.async_copy(obuf, out_hbm.at[q0 + c], osem[ob])
            return ()

        lax.fori_loop(0, seq_per_w // 4, four_seqs, ())
        pltpu.make_async_copy(obufs.at[0], out_hbm.at[q0], osem0).wait()
        pltpu.make_async_copy(obufs.at[1], out_hbm.at[q0], osem1).wait()

    out = emb_ln(ids, item_table, pos_table, ln_gamma, ln_beta)
    return out
